# baseline (speedup 1.0000x reference)
import functools

import jax
import jax.numpy as jnp
from jax.experimental import pallas as pl
from jax.experimental.pallas import tpu as pltpu

_BLOCK = 8192


_KCHUNK = 128


def _nearest_body(x_ref, cneg2_ref, c2_ref, idx_ref, dist_ref):
    x = x_ref[...]
    k = cneg2_ref.shape[0]
    m = None
    am = None
    for c in range(k // _KCHUNK):
        cc = cneg2_ref[c * _KCHUNK:(c + 1) * _KCHUNK, :]
        c2c = c2_ref[c * _KCHUNK:(c + 1) * _KCHUNK, :]
        dpc = jax.lax.dot_general(
            cc, x, (((1,), (1,)), ((), ())),
            preferred_element_type=jnp.float32) + c2c
        mc = jnp.min(dpc, axis=0)
        amc = jnp.argmin(dpc, axis=0).astype(jnp.int32)
        if m is None:
            m, am = mc, amc
        else:
            upd = mc < m
            am = jnp.where(upd, amc + c * _KCHUNK, am)
            m = jnp.minimum(m, mc)
    xx = x * x
    ones = jnp.ones((1, x.shape[1]), jnp.float32)
    x2 = jax.lax.dot_general(
        ones, xx, (((1,), (1,)), ((), ())),
        preferred_element_type=jnp.float32)[0]
    idx_ref[...] = am
    dist_ref[...] = jnp.sqrt(jnp.maximum(x2 + m, 0.0))


@functools.partial(jax.jit, static_argnames=())
def kernel(X, codebook):
    n, d = X.shape
    k = codebook.shape[0]
    grid = n // _BLOCK
    cneg2 = -2.0 * codebook
    c2 = jnp.sum(codebook * codebook, axis=1)[:, None]
    idx, dist = pl.pallas_call(
        _nearest_body,
        grid=(grid,),
        in_specs=[
            pl.BlockSpec((_BLOCK, d), lambda i: (i, 0)),
            pl.BlockSpec((k, d), lambda i: (0, 0)),
            pl.BlockSpec((k, 1), lambda i: (0, 0)),
        ],
        out_specs=[
            pl.BlockSpec((_BLOCK,), lambda i: (i,)),
            pl.BlockSpec((_BLOCK,), lambda i: (i,)),
        ],
        out_shape=[
            jax.ShapeDtypeStruct((n,), jnp.int32),
            jax.ShapeDtypeStruct((n,), jnp.float32),
        ],
        compiler_params=pltpu.CompilerParams(
            dimension_semantics=("parallel",)),
    )(X, cneg2, c2)
    return (idx, dist)


# speedup vs baseline: 1.2528x; 1.2528x over previous
import functools

import jax
import jax.numpy as jnp
from jax.experimental import pallas as pl
from jax.experimental.pallas import tpu as pltpu

_BLOCK = 8192


def _nearest_body(x_ref, cneg2_ref, c2_ref, idx_ref, dist_ref):
    x = x_ref[...]
    b, d = x.shape
    cneg2 = cneg2_ref[...]
    c2 = c2_ref[...]
    k = cneg2.shape[0]
    dots = jax.lax.dot_general(
        cneg2, x, (((1,), (1,)), ((), ())),
        preferred_element_type=jnp.float32)

    m = dots[0:8, :] + c2[0:8, :]
    slab = jnp.zeros((8, b), jnp.int32)
    for i in range(1, k // 8):
        v = dots[8 * i:8 * (i + 1), :] + c2[8 * i:8 * (i + 1), :]
        cmp = v < m
        m = jnp.minimum(m, v)
        slab = jnp.where(cmp, i, slab)

    srow = jax.lax.broadcasted_iota(jnp.int32, (8, b), 0)
    ks = slab * 8 + srow
    m8 = jnp.min(m, axis=0, keepdims=True)
    ks_masked = jnp.where(m == m8, ks, k)
    am = jnp.min(ks_masked, axis=0)

    xx = x * x
    ones = jnp.ones((1, d), jnp.float32)
    x2 = jax.lax.dot_general(
        ones, xx, (((1,), (1,)), ((), ())),
        preferred_element_type=jnp.float32)[0]
    idx_ref[...] = am
    dist_ref[...] = jnp.sqrt(jnp.maximum(x2 + m8[0], 0.0))


@functools.partial(jax.jit, static_argnames=())
def kernel(X, codebook):
    n, d = X.shape
    k = codebook.shape[0]
    grid = n // _BLOCK
    cneg2 = -2.0 * codebook
    c2 = jnp.sum(codebook * codebook, axis=1)[:, None]
    idx, dist = pl.pallas_call(
        _nearest_body,
        grid=(grid,),
        in_specs=[
            pl.BlockSpec((_BLOCK, d), lambda i: (i, 0)),
            pl.BlockSpec((k, d), lambda i: (0, 0)),
            pl.BlockSpec((k, 1), lambda i: (0, 0)),
        ],
        out_specs=[
            pl.BlockSpec((_BLOCK,), lambda i: (i,)),
            pl.BlockSpec((_BLOCK,), lambda i: (i,)),
        ],
        out_shape=[
            jax.ShapeDtypeStruct((n,), jnp.int32),
            jax.ShapeDtypeStruct((n,), jnp.float32),
        ],
        compiler_params=pltpu.CompilerParams(
            dimension_semantics=("arbitrary",)),
    )(X, cneg2, c2)
    return (idx, dist)
